# baseline (speedup 1.0000x reference)
import jax
import jax.numpy as jnp
from jax.experimental import pallas as pl

_NUM_POSITION = 64
_DECLAY = 0.4
_NUM_HIERARCHY = 3
_LAST_P = int(_NUM_POSITION * _DECLAY ** (_NUM_HIERARCHY - 1))


def _reduce_kernel(x_ref, o_ref):
    m = pl.program_id(1)
    n_m = pl.num_programs(1)
    s = jnp.sum(x_ref[0, 0], axis=(1, 2)) * (1.0 / (n_m * _LAST_P))

    @pl.when(m == 0)
    def _init():
        o_ref[0, 0, :] = s

    @pl.when(m != 0)
    def _acc():
        o_ref[0, 0, :] += s


def kernel(x, protos0, protos1, protos2):
    N, M, C, T, V = x.shape
    out = pl.pallas_call(
        _reduce_kernel,
        grid=(N, M),
        in_specs=[pl.BlockSpec((1, 1, C, T, V), lambda n, m: (n, m, 0, 0, 0))],
        out_specs=pl.BlockSpec((1, 1, C), lambda n, m: (n, 0, 0)),
        out_shape=jax.ShapeDtypeStruct((N, 1, C), jnp.float32),
    )(x)
    return out.reshape(N, C)


# speedup vs baseline: 1.2664x; 1.2664x over previous
import jax
import jax.numpy as jnp
from jax.experimental import pallas as pl

_NUM_POSITION = 64
_DECLAY = 0.4
_NUM_HIERARCHY = 3
_LAST_P = int(_NUM_POSITION * _DECLAY ** (_NUM_HIERARCHY - 1))


def _reduce_kernel(x_ref, o_ref):
    o_ref[0, :, 0] = jnp.sum(x_ref[0], axis=1)


def kernel(x, protos0, protos1, protos2):
    N, M, C, T, V = x.shape
    xr = x.reshape(N, M * C, T * V)
    part = pl.pallas_call(
        _reduce_kernel,
        grid=(N,),
        in_specs=[pl.BlockSpec((1, M * C, T * V), lambda i: (i, 0, 0))],
        out_specs=pl.BlockSpec((1, M * C, 1), lambda i: (i, 0, 0)),
        out_shape=jax.ShapeDtypeStruct((N, M * C, 1), jnp.float32),
    )(xr)
    part = part.reshape(N, M, C)
    return part.sum(axis=1) * (1.0 / (M * _LAST_P))
